# baseline (speedup 1.0000x reference)
import jax
import jax.numpy as jnp
from jax.experimental import pallas as pl
from jax.experimental.pallas import tpu as pltpu

_T = 256
_KS = 2


def _mm_kernel(x_ref, nw_ref, ow_ref, o1_ref, o2_ref):
    h = pl.program_id(1)
    x = x_ref[...]
    dn = (((1,), (1,)), ((), ()))
    p1 = jax.lax.dot_general(x, nw_ref[...], dn, preferred_element_type=jnp.float32)
    p2 = jax.lax.dot_general(x, ow_ref[...], dn, preferred_element_type=jnp.float32)

    @pl.when(h == 0)
    def _init():
        o1_ref[...] = p1
        o2_ref[...] = p2

    @pl.when(h != 0)
    def _acc():
        o1_ref[...] += p1
        o2_ref[...] += p2


def kernel(x, new_weight, orig_weight):
    M, K = x.shape
    N = new_weight.shape[0]
    out1, out2 = pl.pallas_call(
        _mm_kernel,
        grid=(N // _T, _KS),
        in_specs=[
            pl.BlockSpec((M, K // _KS), lambda j, h: (0, h)),
            pl.BlockSpec((_T, K // _KS), lambda j, h: (j, h)),
            pl.BlockSpec((_T, K // _KS), lambda j, h: (j, h)),
        ],
        out_specs=[
            pl.BlockSpec((M, _T), lambda j, h: (0, j)),
            pl.BlockSpec((M, _T), lambda j, h: (0, j)),
        ],
        out_shape=[
            jax.ShapeDtypeStruct((M, N), jnp.float32),
            jax.ShapeDtypeStruct((M, N), jnp.float32),
        ],
        compiler_params=pltpu.CompilerParams(
            dimension_semantics=("arbitrary", "arbitrary")),
    )(x, new_weight, orig_weight)
    return (out1, out2)


# speedup vs baseline: 1.1157x; 1.1157x over previous
import jax
import jax.numpy as jnp
from jax.experimental import pallas as pl
from jax.experimental.pallas import tpu as pltpu

_T = 512


def _mm_kernel(x_ref, nw1_ref, nw2_ref, ow1_ref, ow2_ref, o1_ref, o2_ref):
    x = x_ref[...]
    kh = x.shape[1] // 2
    xa, xb = x[:, :kh], x[:, kh:]
    dn = (((1,), (1,)), ((), ()))
    o1_ref[...] = (
        jax.lax.dot_general(xa, nw1_ref[...], dn, preferred_element_type=jnp.float32)
        + jax.lax.dot_general(xb, nw2_ref[...], dn, preferred_element_type=jnp.float32))
    o2_ref[...] = (
        jax.lax.dot_general(xa, ow1_ref[...], dn, preferred_element_type=jnp.float32)
        + jax.lax.dot_general(xb, ow2_ref[...], dn, preferred_element_type=jnp.float32))


def kernel(x, new_weight, orig_weight):
    M, K = x.shape
    N = new_weight.shape[0]
    out1, out2 = pl.pallas_call(
        _mm_kernel,
        grid=(N // _T,),
        in_specs=[
            pl.BlockSpec((M, K), lambda j: (0, 0)),
            pl.BlockSpec((_T, K // 2), lambda j: (j, 0)),
            pl.BlockSpec((_T, K // 2), lambda j: (j, 1)),
            pl.BlockSpec((_T, K // 2), lambda j: (j, 0)),
            pl.BlockSpec((_T, K // 2), lambda j: (j, 1)),
        ],
        out_specs=[
            pl.BlockSpec((M, _T), lambda j: (0, j)),
            pl.BlockSpec((M, _T), lambda j: (0, j)),
        ],
        out_shape=[
            jax.ShapeDtypeStruct((M, N), jnp.float32),
            jax.ShapeDtypeStruct((M, N), jnp.float32),
        ],
        compiler_params=pltpu.CompilerParams(
            dimension_semantics=("arbitrary",)),
    )(x, new_weight, new_weight, orig_weight, orig_weight)
    return (out1, out2)


# speedup vs baseline: 1.1608x; 1.0404x over previous
import jax
import jax.numpy as jnp
from jax.experimental import pallas as pl
from jax.experimental.pallas import tpu as pltpu

_T = 256


def _mm_kernel(x_ref, nw1, nw2, nw3, nw4, ow1, ow2, ow3, ow4, o1_ref, o2_ref):
    x = x_ref[...]
    kq = x.shape[1] // 4
    xs = [x[:, i * kq:(i + 1) * kq] for i in range(4)]
    dn = (((1,), (1,)), ((), ()))
    nws = [nw1, nw2, nw3, nw4]
    ows = [ow1, ow2, ow3, ow4]
    o1_ref[...] = sum(
        jax.lax.dot_general(xs[i], nws[i][...], dn, preferred_element_type=jnp.float32)
        for i in range(4))
    o2_ref[...] = sum(
        jax.lax.dot_general(xs[i], ows[i][...], dn, preferred_element_type=jnp.float32)
        for i in range(4))


def kernel(x, new_weight, orig_weight):
    M, K = x.shape
    N = new_weight.shape[0]
    wspec = [pl.BlockSpec((_T, K // 4), (lambda i: (lambda j: (j, i)))(i))
             for i in range(4)]
    out1, out2 = pl.pallas_call(
        _mm_kernel,
        grid=(N // _T,),
        in_specs=[pl.BlockSpec((M, K), lambda j: (0, 0))] + wspec + wspec,
        out_specs=[
            pl.BlockSpec((M, _T), lambda j: (0, j)),
            pl.BlockSpec((M, _T), lambda j: (0, j)),
        ],
        out_shape=[
            jax.ShapeDtypeStruct((M, N), jnp.float32),
            jax.ShapeDtypeStruct((M, N), jnp.float32),
        ],
        compiler_params=pltpu.CompilerParams(
            dimension_semantics=("arbitrary",)),
    )(x, new_weight, new_weight, new_weight, new_weight,
      orig_weight, orig_weight, orig_weight, orig_weight)
    return (out1, out2)


# speedup vs baseline: 1.1836x; 1.0196x over previous
import jax
import jax.numpy as jnp
from jax.experimental import pallas as pl
from jax.experimental.pallas import tpu as pltpu

_T = 256
_S = 8


def _mm_kernel(*refs):
    x_ref = refs[0]
    nws = refs[1:1 + _S]
    ows = refs[1 + _S:1 + 2 * _S]
    o1_ref, o2_ref = refs[1 + 2 * _S], refs[2 + 2 * _S]
    x = x_ref[...]
    kq = x.shape[1] // _S
    xs = [x[:, i * kq:(i + 1) * kq] for i in range(_S)]
    dn = (((1,), (1,)), ((), ()))
    o1_ref[...] = sum(
        jax.lax.dot_general(xs[i], nws[i][...], dn, preferred_element_type=jnp.float32)
        for i in range(_S))
    o2_ref[...] = sum(
        jax.lax.dot_general(xs[i], ows[i][...], dn, preferred_element_type=jnp.float32)
        for i in range(_S))


def kernel(x, new_weight, orig_weight):
    M, K = x.shape
    N = new_weight.shape[0]
    wspec = [pl.BlockSpec((_T, K // _S), (lambda i: (lambda j: (j, i)))(i))
             for i in range(_S)]
    out1, out2 = pl.pallas_call(
        _mm_kernel,
        grid=(N // _T,),
        in_specs=[pl.BlockSpec((M, K), lambda j: (0, 0))] + wspec + wspec,
        out_specs=[
            pl.BlockSpec((M, _T), lambda j: (0, j)),
            pl.BlockSpec((M, _T), lambda j: (0, j)),
        ],
        out_shape=[
            jax.ShapeDtypeStruct((M, N), jnp.float32),
            jax.ShapeDtypeStruct((M, N), jnp.float32),
        ],
        compiler_params=pltpu.CompilerParams(
            dimension_semantics=("arbitrary",)),
    )(x, *([new_weight] * _S), *([orig_weight] * _S))
    return (out1, out2)


# speedup vs baseline: 1.1885x; 1.0042x over previous
import jax
import jax.numpy as jnp
from jax.experimental import pallas as pl
from jax.experimental.pallas import tpu as pltpu

_T = 256
_S = 16


def _mm_kernel(*refs):
    x_ref = refs[0]
    nws = refs[1:1 + _S]
    ows = refs[1 + _S:1 + 2 * _S]
    o1_ref, o2_ref = refs[1 + 2 * _S], refs[2 + 2 * _S]
    x = x_ref[...]
    kq = x.shape[1] // _S
    xs = [x[:, i * kq:(i + 1) * kq] for i in range(_S)]
    dn = (((1,), (1,)), ((), ()))
    o1_ref[...] = sum(
        jax.lax.dot_general(xs[i], nws[i][...], dn, preferred_element_type=jnp.float32)
        for i in range(_S))
    o2_ref[...] = sum(
        jax.lax.dot_general(xs[i], ows[i][...], dn, preferred_element_type=jnp.float32)
        for i in range(_S))


def kernel(x, new_weight, orig_weight):
    M, K = x.shape
    N = new_weight.shape[0]
    wspec = [pl.BlockSpec((_T, K // _S), (lambda i: (lambda j: (j, i)))(i))
             for i in range(_S)]
    out1, out2 = pl.pallas_call(
        _mm_kernel,
        grid=(N // _T,),
        in_specs=[pl.BlockSpec((M, K), lambda j: (0, 0))] + wspec + wspec,
        out_specs=[
            pl.BlockSpec((M, _T), lambda j: (0, j)),
            pl.BlockSpec((M, _T), lambda j: (0, j)),
        ],
        out_shape=[
            jax.ShapeDtypeStruct((M, N), jnp.float32),
            jax.ShapeDtypeStruct((M, N), jnp.float32),
        ],
        compiler_params=pltpu.CompilerParams(
            dimension_semantics=("arbitrary",)),
    )(x, *([new_weight] * _S), *([orig_weight] * _S))
    return (out1, out2)
